# baseline (speedup 1.0000x reference)
import functools

import jax
import jax.numpy as jnp
from jax import lax
from jax.experimental import pallas as pl
from jax.experimental.pallas import tpu as pltpu
from jax.experimental.pallas import tpu_sc as plsc

NC = 2
NS = 16
L = 16
NW = NC * NS

BN_EPS = 1e-5


def _sc_pool(idx_flat, emb_table, lin_flat, B, F, D, CH):
    b_per_w = B // NW
    n_chunks = b_per_w // CH
    GF = CH * F

    mesh = plsc.VectorSubcoreMesh(core_axis_name="c", subcore_axis_name="s")

    @functools.partial(
        pl.kernel,
        out_type=[
            jax.ShapeDtypeStruct((B, D), jnp.float32),
            jax.ShapeDtypeStruct((B,), jnp.float32),
        ],
        mesh=mesh,
        scratch_types=[
            pltpu.VMEM((GF,), jnp.int32),
            pltpu.VMEM((GF, D), jnp.float32),
            pltpu.VMEM((GF,), jnp.float32),
            pltpu.VMEM((CH, D), jnp.float32),
            pltpu.VMEM((CH,), jnp.float32),
            pltpu.SemaphoreType.DMA,
            pltpu.SemaphoreType.DMA,
        ],
        compiler_params=pltpu.CompilerParams(use_tc_tiling_on_sc=False),
    )
    def k(idx_hbm, emb_hbm, lin_hbm, cross_out, lin_out,
          idx_v, rows_v, lin_v, cross_v, linsum_v, sem_e, sem_l):
        wid = lax.axis_index("s") * NC + lax.axis_index("c")
        base = wid * b_per_w

        def chunk_body(c, _):
            row0 = base + c * CH
            pltpu.sync_copy(idx_hbm.at[pl.ds(row0 * F, GF)], idx_v)
            ce = pltpu.async_copy(emb_hbm.at[idx_v], rows_v, sem_e)
            cl = pltpu.async_copy(lin_hbm.at[idx_v], lin_v, sem_l)
            ce.wait()

            def row_body(i, _):
                s = rows_v[i]
                q = s * s
                for f in range(1, F):
                    r = rows_v[f * CH + i]
                    s = s + r
                    q = q + r * r
                cross_v[i] = 0.5 * (s * s - q)
                return 0

            lax.fori_loop(0, CH, row_body, 0, unroll=False)
            cl.wait()

            def grp_body(g, _):
                acc = lin_v[pl.ds(g * L, L)]
                for f in range(1, F):
                    acc = acc + lin_v[pl.ds(f * CH + g * L, L)]
                linsum_v[pl.ds(g * L, L)] = acc
                return 0

            lax.fori_loop(0, CH // L, grp_body, 0, unroll=False)

            pltpu.sync_copy(cross_v, cross_out.at[pl.ds(row0, CH)])
            pltpu.sync_copy(linsum_v, lin_out.at[pl.ds(row0, CH)])
            return 0

        lax.fori_loop(0, n_chunks, chunk_body, 0, unroll=False)

    return k(idx_flat, emb_table, lin_flat)


def _tc_mlp(cross, lin, W1f, b1f, W2t, c0, B, D, H):
    BS = 2048

    def body(cross_ref, lin_ref, w1_ref, b1_ref, w2_ref, c_ref, out_ref):
        h = jnp.dot(cross_ref[...], w1_ref[...],
                    preferred_element_type=jnp.float32) + b1_ref[...]
        h = jnp.maximum(h, 0.0)
        o = jnp.sum(h * w2_ref[...], axis=1)
        out_ref[...] = o + lin_ref[...] + c_ref[0]

    return pl.pallas_call(
        body,
        grid=(B // BS,),
        in_specs=[
            pl.BlockSpec((BS, D), lambda i: (i, 0)),
            pl.BlockSpec((BS,), lambda i: (i,)),
            pl.BlockSpec((D, H), lambda i: (0, 0)),
            pl.BlockSpec((1, H), lambda i: (0, 0)),
            pl.BlockSpec((1, H), lambda i: (0, 0)),
            pl.BlockSpec(memory_space=pltpu.SMEM),
        ],
        out_specs=pl.BlockSpec((BS,), lambda i: (i,)),
        out_shape=jax.ShapeDtypeStruct((B,), jnp.float32),
    )(cross, lin, W1f, b1f, W2t, c0)


def kernel(x, emb_table, lin_table, lin_bias, bn_fm_gamma, bn_fm_beta,
           W1, b1, bn1_gamma, bn1_beta, W2, b2):
    B, F = x.shape
    V, D = emb_table.shape
    H = W1.shape[1]

    offsets = (V // F) * jnp.arange(F, dtype=x.dtype)
    CH = 128
    idx = (x + offsets[None, :]).reshape(B // CH, CH, F)
    idx = idx.transpose(0, 2, 1).reshape(-1)

    cross, lin = _sc_pool(idx, emb_table, lin_table.reshape(-1), B, F, D,
                          CH=CH)

    inv = 1.0 / jnp.sqrt(1.0 + BN_EPS)
    g0 = bn_fm_gamma * inv
    g1 = bn1_gamma * inv
    W1f = (g0[:, None] * W1) * g1[None, :]
    b1f = ((bn_fm_beta @ W1 + b1) * g1 + bn1_beta)[None, :]
    W2t = W2.reshape(1, H)
    c0 = (b2 + lin_bias).reshape(1)

    return _tc_mlp(cross, lin, W1f, b1f, W2t, c0, B, D, H)


# speedup vs baseline: 1.0055x; 1.0055x over previous
import functools

import jax
import jax.numpy as jnp
from jax import lax
from jax.experimental import pallas as pl
from jax.experimental.pallas import tpu as pltpu
from jax.experimental.pallas import tpu_sc as plsc

NC = 2
NS = 16
L = 16
NW = NC * NS

BN_EPS = 1e-5


def _sc_pool(idx_flat, emb_table, lin_flat, B, F, D, CH):
    b_per_w = B // NW
    n_chunks = b_per_w // CH
    GF = CH * F

    mesh = plsc.VectorSubcoreMesh(core_axis_name="c", subcore_axis_name="s")

    @functools.partial(
        pl.kernel,
        out_type=[
            jax.ShapeDtypeStruct((B, D), jnp.float32),
            jax.ShapeDtypeStruct((B,), jnp.float32),
        ],
        mesh=mesh,
        scratch_types=[
            pltpu.VMEM((F, CH), jnp.int32),
            pltpu.VMEM((GF, D), jnp.float32),
            pltpu.VMEM((GF,), jnp.float32),
            pltpu.VMEM((CH, D), jnp.float32),
            pltpu.VMEM((CH,), jnp.float32),
            pltpu.SemaphoreType.DMA,
            pltpu.SemaphoreType.DMA,
        ],
        compiler_params=pltpu.CompilerParams(use_tc_tiling_on_sc=False),
    )
    def k(idx_hbm, emb_hbm, lin_hbm, cross_out, lin_out,
          idx_v, rows_v, lin_v, cross_v, linsum_v, sem_e, sem_l):
        wid = lax.axis_index("s") * NC + lax.axis_index("c")
        base = wid * b_per_w

        def chunk_body(c, _):
            row0 = base + c * CH
            pltpu.sync_copy(idx_hbm.at[:, pl.ds(row0, CH)], idx_v)
            emb_cps = [
                pltpu.async_copy(emb_hbm.at[idx_v.at[f]],
                                 rows_v.at[pl.ds(f * CH, CH)], sem_e)
                for f in range(F)
            ]
            lin_cps = [
                pltpu.async_copy(lin_hbm.at[idx_v.at[f]],
                                 lin_v.at[pl.ds(f * CH, CH)], sem_l)
                for f in range(F)
            ]
            for cp in emb_cps:
                cp.wait()

            def row_body(i, _):
                s = rows_v[i]
                q = s * s
                for f in range(1, F):
                    r = rows_v[f * CH + i]
                    s = s + r
                    q = q + r * r
                cross_v[i] = 0.5 * (s * s - q)
                return 0

            lax.fori_loop(0, CH, row_body, 0, unroll=False)
            for cp in lin_cps:
                cp.wait()

            def grp_body(g, _):
                acc = lin_v[pl.ds(g * L, L)]
                for f in range(1, F):
                    acc = acc + lin_v[pl.ds(f * CH + g * L, L)]
                linsum_v[pl.ds(g * L, L)] = acc
                return 0

            lax.fori_loop(0, CH // L, grp_body, 0, unroll=False)

            pltpu.sync_copy(cross_v, cross_out.at[pl.ds(row0, CH)])
            pltpu.sync_copy(linsum_v, lin_out.at[pl.ds(row0, CH)])
            return 0

        lax.fori_loop(0, n_chunks, chunk_body, 0, unroll=False)

    return k(idx_flat, emb_table, lin_flat)


def _tc_mlp(cross, lin, W1f, b1f, W2t, c0, B, D, H):
    BS = 2048

    def body(cross_ref, lin_ref, w1_ref, b1_ref, w2_ref, c_ref, out_ref):
        h = jnp.dot(cross_ref[...], w1_ref[...],
                    preferred_element_type=jnp.float32) + b1_ref[...]
        h = jnp.maximum(h, 0.0)
        o = jnp.sum(h * w2_ref[...], axis=1)
        out_ref[...] = o + lin_ref[...] + c_ref[0]

    return pl.pallas_call(
        body,
        grid=(B // BS,),
        in_specs=[
            pl.BlockSpec((BS, D), lambda i: (i, 0)),
            pl.BlockSpec((BS,), lambda i: (i,)),
            pl.BlockSpec((D, H), lambda i: (0, 0)),
            pl.BlockSpec((1, H), lambda i: (0, 0)),
            pl.BlockSpec((1, H), lambda i: (0, 0)),
            pl.BlockSpec(memory_space=pltpu.SMEM),
        ],
        out_specs=pl.BlockSpec((BS,), lambda i: (i,)),
        out_shape=jax.ShapeDtypeStruct((B,), jnp.float32),
    )(cross, lin, W1f, b1f, W2t, c0)


def kernel(x, emb_table, lin_table, lin_bias, bn_fm_gamma, bn_fm_beta,
           W1, b1, bn1_gamma, bn1_beta, W2, b2):
    B, F = x.shape
    V, D = emb_table.shape
    H = W1.shape[1]

    offsets = (V // F) * jnp.arange(F, dtype=x.dtype)
    idx_fm = x.T + offsets[:, None]

    cross, lin = _sc_pool(idx_fm, emb_table, lin_table.reshape(-1), B, F, D,
                          CH=256)

    inv = 1.0 / jnp.sqrt(1.0 + BN_EPS)
    g0 = bn_fm_gamma * inv
    g1 = bn1_gamma * inv
    W1f = (g0[:, None] * W1) * g1[None, :]
    b1f = ((bn_fm_beta @ W1 + b1) * g1 + bn1_beta)[None, :]
    W2t = W2.reshape(1, H)
    c0 = (b2 + lin_bias).reshape(1)

    return _tc_mlp(cross, lin, W1f, b1f, W2t, c0, B, D, H)
